# Initial kernel scaffold; baseline (speedup 1.0000x reference)
import jax
import jax.numpy as jnp
from jax.experimental import pallas as pl


def kernel(x, edge_index, e, Wk0, bk0, Wk1, bk1, Wk2, bk2, Wu0, bu0, Wu1, bu1, Wu2, bu2, WencK, bencK, WencP1, bencP1, WencP2, bencP2):
    raise NotImplementedError("write your pallas kernel here")


# speedup vs baseline: 3.4865x; 3.4865x over previous
import functools

import jax
import jax.numpy as jnp
from jax import lax
from jax.experimental import pallas as pl
from jax.experimental.pallas import tpu as pltpu
from jax.experimental.pallas import tpu_sc as plsc

N_NODES = 10000
N_EDGES = 320000
D = 128

NC = 2
NS = 16
NW = NC * NS
EPW = N_EDGES // NW
CH = 80
NCHUNK = EPW // CH
N_PAD = 10240
STRIPE = N_PAD // NS

_MESH = plsc.VectorSubcoreMesh(core_axis_name="c", subcore_axis_name="s")


def _enc_body(x_ref, w_ref, b_ref, h1_ref, h21_ref, h22_ref):
    h = jnp.dot(x_ref[...], w_ref[...], preferred_element_type=jnp.float32)
    h = h + b_ref[...]
    h1_ref[...] = h[:, :D]
    h21_ref[...] = h[:, D:2 * D]
    h22_ref[...] = h[:, 2 * D:]


def _encode(x, w_enc, b_enc):
    rows = 2000
    grid = (N_NODES // rows,)
    out = jax.ShapeDtypeStruct((N_NODES, D), jnp.float32)
    return pl.pallas_call(
        _enc_body,
        grid=grid,
        in_specs=[
            pl.BlockSpec((rows, D), lambda i: (i, 0)),
            pl.BlockSpec((D, 3 * D), lambda i: (0, 0)),
            pl.BlockSpec((1, 3 * D), lambda i: (0, 0)),
        ],
        out_specs=[
            pl.BlockSpec((rows, D), lambda i: (i, 0)),
            pl.BlockSpec((rows, D), lambda i: (i, 0)),
            pl.BlockSpec((rows, D), lambda i: (i, 0)),
        ],
        out_shape=[out, out, out],
    )(x, w_enc, b_enc)


@functools.partial(
    pl.kernel,
    out_type=(
        jax.ShapeDtypeStruct((NC * N_PAD, D), jnp.float32),
        jax.ShapeDtypeStruct((N_EDGES, D), jnp.float32),
    ),
    mesh=_MESH,
    scratch_types=[
        pltpu.VMEM((CH,), jnp.int32),
        pltpu.VMEM((CH,), jnp.int32),
        pltpu.VMEM((CH, D), jnp.float32),
        pltpu.VMEM((CH, D), jnp.float32),
        pltpu.VMEM_SHARED((N_PAD, D), jnp.float32),
        pltpu.SemaphoreType.DMA,
    ],
)
def _sc1(src_hbm, dst_hbm, h1_hbm, h21_hbm, h22_hbm, z_hbm,
         agg_hbm, s_hbm, idx_s, idx_d, rows, srows, aggsh, sem):
    c = lax.axis_index("c")
    sidx = lax.axis_index("s")
    wid = sidx * NC + c
    tid = sidx

    pltpu.sync_copy(z_hbm, aggsh.at[pl.ds(tid * STRIPE, STRIPE)])
    plsc.subcore_barrier()

    def body(i, carry):
        base = wid * EPW + i * CH
        pltpu.sync_copy(src_hbm.at[pl.ds(base, CH)], idx_s)
        pltpu.sync_copy(dst_hbm.at[pl.ds(base, CH)], idx_d)
        pltpu.async_copy(h1_hbm.at[idx_s], rows, sem).wait()
        pltpu.sync_copy(rows, aggsh.at[idx_d], add=True)
        pltpu.async_copy(h21_hbm.at[idx_s], srows, sem).wait()
        pltpu.async_copy(h22_hbm.at[idx_d], srows, sem, add=True).wait()
        pltpu.sync_copy(srows, s_hbm.at[pl.ds(base, CH)])
        return carry

    lax.fori_loop(0, NCHUNK, body, 0)

    plsc.subcore_barrier()
    pltpu.sync_copy(aggsh.at[pl.ds(tid * STRIPE, STRIPE)],
                    agg_hbm.at[pl.ds(c * N_PAD + tid * STRIPE, STRIPE)])


def _umlp_body(s_ref, w0, b0, w1, b1, w2, b2, u_ref):
    h = jnp.tanh(jnp.dot(s_ref[...], w0[...],
                         preferred_element_type=jnp.float32) + b0[...])
    h = jnp.maximum(jnp.dot(h, w1[...],
                            preferred_element_type=jnp.float32) + b1[...], 0.0)
    u_ref[...] = jnp.dot(h, w2[...],
                         preferred_element_type=jnp.float32) + b2[...]


def _umlp(s, w0, b0, w1, b1, w2, b2):
    rows = 2000
    grid = (N_EDGES // rows,)
    wspec = pl.BlockSpec((D, D), lambda i: (0, 0))
    bspec = pl.BlockSpec((1, D), lambda i: (0, 0))
    return pl.pallas_call(
        _umlp_body,
        grid=grid,
        in_specs=[pl.BlockSpec((rows, D), lambda i: (i, 0)),
                  wspec, bspec, wspec, bspec, wspec, bspec],
        out_specs=pl.BlockSpec((rows, D), lambda i: (i, 0)),
        out_shape=jax.ShapeDtypeStruct((N_EDGES, D), jnp.float32),
    )(s, w0, b0, w1, b1, w2, b2)


@functools.partial(
    pl.kernel,
    out_type=jax.ShapeDtypeStruct((NC * N_PAD, D), jnp.float32),
    mesh=_MESH,
    scratch_types=[
        pltpu.VMEM((CH,), jnp.int32),
        pltpu.VMEM((CH, D), jnp.float32),
        pltpu.VMEM_SHARED((N_PAD, D), jnp.float32),
    ],
)
def _sc2(src_hbm, u_hbm, z_hbm, usum_hbm, idx_s, rows, ussh):
    c = lax.axis_index("c")
    sidx = lax.axis_index("s")
    wid = sidx * NC + c
    tid = sidx

    pltpu.sync_copy(z_hbm, ussh.at[pl.ds(tid * STRIPE, STRIPE)])
    plsc.subcore_barrier()

    def body(i, carry):
        base = wid * EPW + i * CH
        pltpu.sync_copy(src_hbm.at[pl.ds(base, CH)], idx_s)
        pltpu.sync_copy(u_hbm.at[pl.ds(base, CH)], rows)
        pltpu.sync_copy(rows, ussh.at[idx_s], add=True)
        return carry

    lax.fori_loop(0, NCHUNK, body, 0)

    plsc.subcore_barrier()
    pltpu.sync_copy(ussh.at[pl.ds(tid * STRIPE, STRIPE)],
                    usum_hbm.at[pl.ds(c * N_PAD + tid * STRIPE, STRIPE)])


def _kdot_body(agg_ref, usum_ref, w0, b0, w1, b1, w2, b2, out_ref):
    a = agg_ref[0] + agg_ref[1]
    us = usum_ref[0] + usum_ref[1]
    h = jnp.tanh(jnp.dot(a, w0[...],
                         preferred_element_type=jnp.float32) + b0[...])
    h = jnp.maximum(jnp.dot(h, w1[...],
                            preferred_element_type=jnp.float32) + b1[...], 0.0)
    k = jnp.dot(h, w2[...], preferred_element_type=jnp.float32) + b2[...]
    part = jnp.sum(k * us).reshape(1, 1)

    @pl.when(pl.program_id(0) == 0)
    def _():
        out_ref[...] = jnp.zeros((1, 1), jnp.float32)

    out_ref[...] += part


def _kdot(agg, usum, w0, b0, w1, b1, w2, b2):
    rows = 2048
    grid = (N_PAD // rows,)
    wspec = pl.BlockSpec((D, D), lambda i: (0, 0))
    bspec = pl.BlockSpec((1, D), lambda i: (0, 0))
    out = pl.pallas_call(
        _kdot_body,
        grid=grid,
        in_specs=[pl.BlockSpec((NC, rows, D), lambda i: (0, i, 0)),
                  pl.BlockSpec((NC, rows, D), lambda i: (0, i, 0)),
                  wspec, bspec, wspec, bspec, wspec, bspec],
        out_specs=pl.BlockSpec((1, 1), lambda i: (0, 0)),
        out_shape=jax.ShapeDtypeStruct((1, 1), jnp.float32),
    )(agg, usum, w0, b0, w1, b1, w2, b2)
    return out[0, 0]


def kernel(x, edge_index, e,
           Wk0, bk0, Wk1, bk1, Wk2, bk2,
           Wu0, bu0, Wu1, bu1, Wu2, bu2,
           WencK, bencK, WencP1, bencP1, WencP2, bencP2):
    src = edge_index[0]
    dst = edge_index[1]

    w_enc = jnp.concatenate([WencK.T, WencP1.T, WencP2.T], axis=1)
    b_enc = jnp.concatenate([bencK, bencP1, bencP2])[None, :]
    h1, h21, h22 = _encode(x, w_enc, b_enc)

    z = jnp.zeros((STRIPE, D), jnp.float32)
    agg, s = _sc1(src, dst, h1, h21, h22, z)

    u = _umlp(s, Wu0.T, bu0[None, :], Wu1.T, bu1[None, :], Wu2.T, bu2[None, :])

    usum = _sc2(src, u, z)

    agg3 = agg.reshape(NC, N_PAD, D)
    usum3 = usum.reshape(NC, N_PAD, D)
    return _kdot(agg3, usum3,
                 Wk0.T, bk0[None, :], Wk1.T, bk1[None, :], Wk2.T, bk2[None, :])


# speedup vs baseline: 4.4799x; 1.2849x over previous
import functools

import jax
import jax.numpy as jnp
from jax import lax
from jax.experimental import pallas as pl
from jax.experimental.pallas import tpu as pltpu
from jax.experimental.pallas import tpu_sc as plsc

N_NODES = 10000
N_EDGES = 320000
D = 128

NC = 2
NS = 16
NW = NC * NS
CH = 64
NCHT = N_EDGES // CH
NPAIRT = NCHT // 2
NPITER = 80
N_PAD = 10240
STRIPE = N_PAD // NS

_MESH = plsc.VectorSubcoreMesh(core_axis_name="c", subcore_axis_name="s")


def _enc_body(x_ref, w_ref, b_ref, h1_ref, h21_ref, h22_ref):
    h = jnp.dot(x_ref[...], w_ref[...], preferred_element_type=jnp.float32)
    h = h + b_ref[...]
    h1_ref[...] = h[:, :D]
    h21_ref[...] = h[:, D:2 * D]
    h22_ref[...] = h[:, 2 * D:]


def _encode(x, w_enc, b_enc):
    rows = 2000
    grid = (N_NODES // rows,)
    out = jax.ShapeDtypeStruct((N_NODES, D), jnp.float32)
    return pl.pallas_call(
        _enc_body,
        grid=grid,
        in_specs=[
            pl.BlockSpec((rows, D), lambda i: (i, 0)),
            pl.BlockSpec((D, 3 * D), lambda i: (0, 0)),
            pl.BlockSpec((1, 3 * D), lambda i: (0, 0)),
        ],
        out_specs=[
            pl.BlockSpec((rows, D), lambda i: (i, 0)),
            pl.BlockSpec((rows, D), lambda i: (i, 0)),
            pl.BlockSpec((rows, D), lambda i: (i, 0)),
        ],
        out_shape=[out, out, out],
    )(x, w_enc, b_enc)


@functools.partial(
    pl.kernel,
    out_type=(
        jax.ShapeDtypeStruct((NC * N_PAD, D), jnp.float32),
        jax.ShapeDtypeStruct((N_EDGES, D), jnp.float32),
    ),
    mesh=_MESH,
    scratch_types=[
        pltpu.VMEM((2, CH), jnp.int32),
        pltpu.VMEM((2, CH), jnp.int32),
        pltpu.VMEM((CH, D), jnp.float32),
        pltpu.VMEM((CH, D), jnp.float32),
        pltpu.VMEM((CH, D), jnp.float32),
        pltpu.VMEM((CH, D), jnp.float32),
        pltpu.VMEM_SHARED((N_PAD, D), jnp.float32),
        pltpu.SemaphoreType.DMA,
        pltpu.SemaphoreType.DMA,
        pltpu.SemaphoreType.DMA,
        pltpu.SemaphoreType.DMA,
        pltpu.SemaphoreType.DMA,
        pltpu.SemaphoreType.DMA,
    ],
)
def _sc1(src_hbm, dst_hbm, h1_hbm, h21_hbm, h22_hbm, z_hbm,
         agg_hbm, s_hbm, idx_s, idx_d, rows_a, rows_b, srows_a, srows_b,
         aggsh, sem_i, sem_g1, sem_g2, sem_g3, sem_sc, sem_w):
    c = lax.axis_index("c")
    sidx = lax.axis_index("s")
    wid = sidx * NC + c
    tid = sidx

    pltpu.sync_copy(z_hbm, aggsh.at[pl.ds(tid * STRIPE, STRIPE)])
    plsc.subcore_barrier()

    def chunk(idx_row, rows, srows, sem_ga, sem_gb, base):
        g1 = pltpu.async_copy(h1_hbm.at[idx_s.at[idx_row]], rows, sem_ga)
        g2 = pltpu.async_copy(h21_hbm.at[idx_s.at[idx_row]], srows, sem_gb)
        g1.wait()
        sc = pltpu.async_copy(rows, aggsh.at[idx_d.at[idx_row]], sem_sc,
                              add=True)
        g2.wait()
        g3 = pltpu.async_copy(h22_hbm.at[idx_d.at[idx_row]], srows, sem_gb,
                              add=True)
        g3.wait()
        w = pltpu.async_copy(srows, s_hbm.at[pl.ds(base, CH)], sem_w)
        return sc, w

    def body(j, carry):
        pid = j * NW + wid

        @pl.when(pid < NPAIRT)
        def _pair():
            i1 = pltpu.async_copy(src_hbm.at[pl.ds(2 * pid, 2)], idx_s, sem_i)
            i2 = pltpu.async_copy(dst_hbm.at[pl.ds(2 * pid, 2)], idx_d, sem_g1)
            i1.wait()
            i2.wait()
            sc_a, w_a = chunk(0, rows_a, srows_a, sem_g1, sem_g2,
                              2 * pid * CH)
            sc_b, w_b = chunk(1, rows_b, srows_b, sem_g1, sem_g3,
                              (2 * pid + 1) * CH)
            sc_a.wait()
            w_a.wait()
            sc_b.wait()
            w_b.wait()

        return carry

    lax.fori_loop(0, NPITER, body, 0)

    plsc.subcore_barrier()
    pltpu.sync_copy(aggsh.at[pl.ds(tid * STRIPE, STRIPE)],
                    agg_hbm.at[pl.ds(c * N_PAD + tid * STRIPE, STRIPE)])


def _umlp_body(s_ref, w0, b0, w1, b1, w2, b2, u_ref):
    h = jnp.tanh(jnp.dot(s_ref[...], w0[...],
                         preferred_element_type=jnp.float32) + b0[...])
    h = jnp.maximum(jnp.dot(h, w1[...],
                            preferred_element_type=jnp.float32) + b1[...], 0.0)
    u_ref[...] = jnp.dot(h, w2[...],
                         preferred_element_type=jnp.float32) + b2[...]


def _umlp(s, w0, b0, w1, b1, w2, b2):
    rows = 2000
    grid = (N_EDGES // rows,)
    wspec = pl.BlockSpec((D, D), lambda i: (0, 0))
    bspec = pl.BlockSpec((1, D), lambda i: (0, 0))
    return pl.pallas_call(
        _umlp_body,
        grid=grid,
        in_specs=[pl.BlockSpec((rows, D), lambda i: (i, 0)),
                  wspec, bspec, wspec, bspec, wspec, bspec],
        out_specs=pl.BlockSpec((rows, D), lambda i: (i, 0)),
        out_shape=jax.ShapeDtypeStruct((N_EDGES, D), jnp.float32),
    )(s, w0, b0, w1, b1, w2, b2)


@functools.partial(
    pl.kernel,
    out_type=jax.ShapeDtypeStruct((NC * N_PAD, D), jnp.float32),
    mesh=_MESH,
    scratch_types=[
        pltpu.VMEM((2, CH), jnp.int32),
        pltpu.VMEM((2, CH), jnp.int32),
        pltpu.VMEM((2 * CH, D), jnp.float32),
        pltpu.VMEM((2 * CH, D), jnp.float32),
        pltpu.VMEM_SHARED((N_PAD, D), jnp.float32),
        pltpu.SemaphoreType.DMA,
        pltpu.SemaphoreType.DMA,
        pltpu.SemaphoreType.DMA,
        pltpu.SemaphoreType.DMA,
    ],
)
def _sc2(src_hbm, u_hbm, z_hbm, usum_hbm, idx_a, idx_b, rows_a, rows_b,
         ussh, sem_i, sem_l, sem_sca, sem_scb):
    c = lax.axis_index("c")
    sidx = lax.axis_index("s")
    wid = sidx * NC + c
    tid = sidx

    pltpu.sync_copy(z_hbm, ussh.at[pl.ds(tid * STRIPE, STRIPE)])
    plsc.subcore_barrier()

    def issue(pid, idx, rows, sem_sc):
        pltpu.sync_copy(src_hbm.at[pl.ds(2 * pid, 2)], idx)
        ld = pltpu.async_copy(u_hbm.at[pl.ds(2 * pid * CH, 2 * CH)], rows,
                              sem_l)
        ld.wait()
        s1 = pltpu.async_copy(rows.at[pl.ds(0, CH)], ussh.at[idx.at[0]],
                              sem_sc, add=True)
        s2 = pltpu.async_copy(rows.at[pl.ds(CH, CH)], ussh.at[idx.at[1]],
                              sem_sc, add=True)
        return s1, s2

    def body(m, carry):
        pid_a = (2 * m) * NW + wid
        pid_b = (2 * m + 1) * NW + wid

        @pl.when(pid_a < NPAIRT)
        def _a():
            sa1, sa2 = issue(pid_a, idx_a, rows_a, sem_sca)

            @pl.when(pid_b < NPAIRT)
            def _b():
                sb1, sb2 = issue(pid_b, idx_b, rows_b, sem_scb)
                sa1.wait()
                sa2.wait()
                sb1.wait()
                sb2.wait()

            @pl.when(jnp.logical_not(pid_b < NPAIRT))
            def _a_only():
                sa1.wait()
                sa2.wait()

        return carry

    lax.fori_loop(0, NPITER // 2, body, 0)

    plsc.subcore_barrier()
    pltpu.sync_copy(ussh.at[pl.ds(tid * STRIPE, STRIPE)],
                    usum_hbm.at[pl.ds(c * N_PAD + tid * STRIPE, STRIPE)])


def _kdot_body(agg_ref, usum_ref, w0, b0, w1, b1, w2, b2, out_ref):
    a = agg_ref[0] + agg_ref[1]
    us = usum_ref[0] + usum_ref[1]
    h = jnp.tanh(jnp.dot(a, w0[...],
                         preferred_element_type=jnp.float32) + b0[...])
    h = jnp.maximum(jnp.dot(h, w1[...],
                            preferred_element_type=jnp.float32) + b1[...], 0.0)
    k = jnp.dot(h, w2[...], preferred_element_type=jnp.float32) + b2[...]
    part = jnp.sum(k * us).reshape(1, 1)

    @pl.when(pl.program_id(0) == 0)
    def _():
        out_ref[...] = jnp.zeros((1, 1), jnp.float32)

    out_ref[...] += part


def _kdot(agg, usum, w0, b0, w1, b1, w2, b2):
    rows = 2048
    grid = (N_PAD // rows,)
    wspec = pl.BlockSpec((D, D), lambda i: (0, 0))
    bspec = pl.BlockSpec((1, D), lambda i: (0, 0))
    out = pl.pallas_call(
        _kdot_body,
        grid=grid,
        in_specs=[pl.BlockSpec((NC, rows, D), lambda i: (0, i, 0)),
                  pl.BlockSpec((NC, rows, D), lambda i: (0, i, 0)),
                  wspec, bspec, wspec, bspec, wspec, bspec],
        out_specs=pl.BlockSpec((1, 1), lambda i: (0, 0)),
        out_shape=jax.ShapeDtypeStruct((1, 1), jnp.float32),
    )(agg, usum, w0, b0, w1, b1, w2, b2)
    return out[0, 0]


def kernel(x, edge_index, e,
           Wk0, bk0, Wk1, bk1, Wk2, bk2,
           Wu0, bu0, Wu1, bu1, Wu2, bu2,
           WencK, bencK, WencP1, bencP1, WencP2, bencP2):
    src = edge_index[0].reshape(NCHT, CH)
    dst = edge_index[1].reshape(NCHT, CH)

    w_enc = jnp.concatenate([WencK.T, WencP1.T, WencP2.T], axis=1)
    b_enc = jnp.concatenate([bencK, bencP1, bencP2])[None, :]
    h1, h21, h22 = _encode(x, w_enc, b_enc)

    z = jnp.zeros((STRIPE, D), jnp.float32)
    agg, s = _sc1(src, dst, h1, h21, h22, z)

    u = _umlp(s, Wu0.T, bu0[None, :], Wu1.T, bu1[None, :], Wu2.T, bu2[None, :])

    usum = _sc2(src, u, z)

    agg3 = agg.reshape(NC, N_PAD, D)
    usum3 = usum.reshape(NC, N_PAD, D)
    return _kdot(agg3, usum3,
                 Wk0.T, bk0[None, :], Wk1.T, bk1[None, :], Wk2.T, bk2[None, :])


# speedup vs baseline: 5.1960x; 1.1599x over previous
import functools

import jax
import jax.numpy as jnp
from jax import lax
from jax.experimental import pallas as pl
from jax.experimental.pallas import tpu as pltpu
from jax.experimental.pallas import tpu_sc as plsc

N_NODES = 10000
N_EDGES = 320000
D = 128

NC = 2
NS = 16
NW = NC * NS
CH = 64
NCHT = N_EDGES // CH
NPAIRT = NCHT // 2
NPITER = 80
N_PAD = 10240
STRIPE = N_PAD // NS

_MESH = plsc.VectorSubcoreMesh(core_axis_name="c", subcore_axis_name="s")


def _enc_body(x_ref, w_ref, b_ref, h1_ref, h21_ref, h22_ref):
    h = jnp.dot(x_ref[...], w_ref[...], preferred_element_type=jnp.float32)
    h = h + b_ref[...]
    h1_ref[...] = h[:, :D]
    h21_ref[...] = h[:, D:2 * D]
    h22_ref[...] = h[:, 2 * D:]


def _encode(x, w_enc, b_enc):
    rows = 2000
    grid = (N_NODES // rows,)
    out = jax.ShapeDtypeStruct((N_NODES, D), jnp.float32)
    return pl.pallas_call(
        _enc_body,
        grid=grid,
        in_specs=[
            pl.BlockSpec((rows, D), lambda i: (i, 0)),
            pl.BlockSpec((D, 3 * D), lambda i: (0, 0)),
            pl.BlockSpec((1, 3 * D), lambda i: (0, 0)),
        ],
        out_specs=[
            pl.BlockSpec((rows, D), lambda i: (i, 0)),
            pl.BlockSpec((rows, D), lambda i: (i, 0)),
            pl.BlockSpec((rows, D), lambda i: (i, 0)),
        ],
        out_shape=[out, out, out],
    )(x, w_enc, b_enc)


@functools.partial(
    pl.kernel,
    out_type=(
        jax.ShapeDtypeStruct((NC * N_PAD, D), jnp.float32),
        jax.ShapeDtypeStruct((N_EDGES, D), jnp.float32),
    ),
    mesh=_MESH,
    scratch_types=[
        pltpu.VMEM((2, CH), jnp.int32),
        pltpu.VMEM((2, CH), jnp.int32),
        pltpu.VMEM((CH, D), jnp.float32),
        pltpu.VMEM((CH, D), jnp.float32),
        pltpu.VMEM((CH, D), jnp.float32),
        pltpu.VMEM((CH, D), jnp.float32),
        pltpu.VMEM_SHARED((N_PAD, D), jnp.float32),
    ] + [pltpu.SemaphoreType.DMA] * 10,
)
def _sc1(src_hbm, dst_hbm, h1_hbm, h21_hbm, h22_hbm, z_hbm,
         agg_hbm, s_hbm, idx_s, idx_d, rows_a, rows_b, srows_a, srows_b,
         aggsh, si1, si2, s1, s2, s3, s4, s5, s6, s7, s8):
    c = lax.axis_index("c")
    sidx = lax.axis_index("s")
    wid = sidx * NC + c
    tid = sidx

    pltpu.sync_copy(z_hbm, aggsh.at[pl.ds(tid * STRIPE, STRIPE)])
    plsc.subcore_barrier()

    def body(j, carry):
        pid = j * NW + wid

        @pl.when(pid < NPAIRT)
        def _pair():
            i1 = pltpu.async_copy(src_hbm.at[pl.ds(2 * pid, 2)], idx_s, si1)
            i2 = pltpu.async_copy(dst_hbm.at[pl.ds(2 * pid, 2)], idx_d, si2)
            i1.wait()
            g1a = pltpu.async_copy(h1_hbm.at[idx_s.at[0]], rows_a, s1)
            g2a = pltpu.async_copy(h21_hbm.at[idx_s.at[0]], srows_a, s2)
            g1b = pltpu.async_copy(h1_hbm.at[idx_s.at[1]], rows_b, s3)
            g2b = pltpu.async_copy(h21_hbm.at[idx_s.at[1]], srows_b, s4)
            i2.wait()
            g1a.wait()
            sca = pltpu.async_copy(rows_a, aggsh.at[idx_d.at[0]], s5,
                                   add=True)
            g2a.wait()
            g3a = pltpu.async_copy(h22_hbm.at[idx_d.at[0]], srows_a, s2,
                                   add=True)
            g1b.wait()
            scb = pltpu.async_copy(rows_b, aggsh.at[idx_d.at[1]], s6,
                                   add=True)
            g2b.wait()
            g3b = pltpu.async_copy(h22_hbm.at[idx_d.at[1]], srows_b, s4,
                                   add=True)
            g3a.wait()
            wa = pltpu.async_copy(srows_a, s_hbm.at[pl.ds(2 * pid * CH, CH)],
                                  s7)
            g3b.wait()
            wb = pltpu.async_copy(srows_b,
                                  s_hbm.at[pl.ds((2 * pid + 1) * CH, CH)], s8)
            sca.wait()
            scb.wait()
            wa.wait()
            wb.wait()

        return carry

    lax.fori_loop(0, NPITER, body, 0)

    plsc.subcore_barrier()
    pltpu.sync_copy(aggsh.at[pl.ds(tid * STRIPE, STRIPE)],
                    agg_hbm.at[pl.ds(c * N_PAD + tid * STRIPE, STRIPE)])


def _umlp_body(s_ref, w0, b0, w1, b1, w2, b2, u_ref):
    h = jnp.tanh(jnp.dot(s_ref[...], w0[...],
                         preferred_element_type=jnp.float32) + b0[...])
    h = jnp.maximum(jnp.dot(h, w1[...],
                            preferred_element_type=jnp.float32) + b1[...], 0.0)
    u_ref[...] = jnp.dot(h, w2[...],
                         preferred_element_type=jnp.float32) + b2[...]


def _umlp(s, w0, b0, w1, b1, w2, b2):
    rows = 2000
    grid = (N_EDGES // rows,)
    wspec = pl.BlockSpec((D, D), lambda i: (0, 0))
    bspec = pl.BlockSpec((1, D), lambda i: (0, 0))
    return pl.pallas_call(
        _umlp_body,
        grid=grid,
        in_specs=[pl.BlockSpec((rows, D), lambda i: (i, 0)),
                  wspec, bspec, wspec, bspec, wspec, bspec],
        out_specs=pl.BlockSpec((rows, D), lambda i: (i, 0)),
        out_shape=jax.ShapeDtypeStruct((N_EDGES, D), jnp.float32),
    )(s, w0, b0, w1, b1, w2, b2)


@functools.partial(
    pl.kernel,
    out_type=jax.ShapeDtypeStruct((NC * N_PAD, D), jnp.float32),
    mesh=_MESH,
    scratch_types=[
        pltpu.VMEM((2, CH), jnp.int32),
        pltpu.VMEM((2, CH), jnp.int32),
        pltpu.VMEM((2 * CH, D), jnp.float32),
        pltpu.VMEM((2 * CH, D), jnp.float32),
        pltpu.VMEM_SHARED((N_PAD, D), jnp.float32),
    ] + [pltpu.SemaphoreType.DMA] * 6,
)
def _sc2(src_hbm, u_hbm, z_hbm, usum_hbm, idx_a, idx_b, rows_a, rows_b,
         ussh, si1, si2, sl1, sl2, sca, scb):
    c = lax.axis_index("c")
    sidx = lax.axis_index("s")
    wid = sidx * NC + c
    tid = sidx

    pltpu.sync_copy(z_hbm, ussh.at[pl.ds(tid * STRIPE, STRIPE)])
    plsc.subcore_barrier()

    def body(m, carry):
        pid_a = (2 * m) * NW + wid
        pid_b = (2 * m + 1) * NW + wid

        @pl.when(pid_a < NPAIRT)
        def _a():
            i_a = pltpu.async_copy(src_hbm.at[pl.ds(2 * pid_a, 2)], idx_a,
                                   si1)
            l_a = pltpu.async_copy(u_hbm.at[pl.ds(2 * pid_a * CH, 2 * CH)],
                                   rows_a, sl1)

            @pl.when(pid_b < NPAIRT)
            def _b():
                i_b = pltpu.async_copy(src_hbm.at[pl.ds(2 * pid_b, 2)],
                                       idx_b, si2)
                l_b = pltpu.async_copy(u_hbm.at[pl.ds(2 * pid_b * CH, 2 * CH)],
                                       rows_b, sl2)
                i_a.wait()
                l_a.wait()
                sa1 = pltpu.async_copy(rows_a.at[pl.ds(0, CH)],
                                       ussh.at[idx_a.at[0]], sca, add=True)
                sa2 = pltpu.async_copy(rows_a.at[pl.ds(CH, CH)],
                                       ussh.at[idx_a.at[1]], sca, add=True)
                i_b.wait()
                l_b.wait()
                sb1 = pltpu.async_copy(rows_b.at[pl.ds(0, CH)],
                                       ussh.at[idx_b.at[0]], scb, add=True)
                sb2 = pltpu.async_copy(rows_b.at[pl.ds(CH, CH)],
                                       ussh.at[idx_b.at[1]], scb, add=True)
                sa1.wait()
                sa2.wait()
                sb1.wait()
                sb2.wait()

            @pl.when(jnp.logical_not(pid_b < NPAIRT))
            def _a_only():
                i_a.wait()
                l_a.wait()
                sa1 = pltpu.async_copy(rows_a.at[pl.ds(0, CH)],
                                       ussh.at[idx_a.at[0]], sca, add=True)
                sa2 = pltpu.async_copy(rows_a.at[pl.ds(CH, CH)],
                                       ussh.at[idx_a.at[1]], sca, add=True)
                sa1.wait()
                sa2.wait()

        return carry

    lax.fori_loop(0, NPITER // 2, body, 0)

    plsc.subcore_barrier()
    pltpu.sync_copy(ussh.at[pl.ds(tid * STRIPE, STRIPE)],
                    usum_hbm.at[pl.ds(c * N_PAD + tid * STRIPE, STRIPE)])


def _kdot_body(agg_ref, usum_ref, w0, b0, w1, b1, w2, b2, out_ref):
    a = agg_ref[0] + agg_ref[1]
    us = usum_ref[0] + usum_ref[1]
    h = jnp.tanh(jnp.dot(a, w0[...],
                         preferred_element_type=jnp.float32) + b0[...])
    h = jnp.maximum(jnp.dot(h, w1[...],
                            preferred_element_type=jnp.float32) + b1[...], 0.0)
    k = jnp.dot(h, w2[...], preferred_element_type=jnp.float32) + b2[...]
    part = jnp.sum(k * us).reshape(1, 1)

    @pl.when(pl.program_id(0) == 0)
    def _():
        out_ref[...] = jnp.zeros((1, 1), jnp.float32)

    out_ref[...] += part


def _kdot(agg, usum, w0, b0, w1, b1, w2, b2):
    rows = 2048
    grid = (N_PAD // rows,)
    wspec = pl.BlockSpec((D, D), lambda i: (0, 0))
    bspec = pl.BlockSpec((1, D), lambda i: (0, 0))
    out = pl.pallas_call(
        _kdot_body,
        grid=grid,
        in_specs=[pl.BlockSpec((NC, rows, D), lambda i: (0, i, 0)),
                  pl.BlockSpec((NC, rows, D), lambda i: (0, i, 0)),
                  wspec, bspec, wspec, bspec, wspec, bspec],
        out_specs=pl.BlockSpec((1, 1), lambda i: (0, 0)),
        out_shape=jax.ShapeDtypeStruct((1, 1), jnp.float32),
    )(agg, usum, w0, b0, w1, b1, w2, b2)
    return out[0, 0]


def kernel(x, edge_index, e,
           Wk0, bk0, Wk1, bk1, Wk2, bk2,
           Wu0, bu0, Wu1, bu1, Wu2, bu2,
           WencK, bencK, WencP1, bencP1, WencP2, bencP2):
    src = edge_index[0].reshape(NCHT, CH)
    dst = edge_index[1].reshape(NCHT, CH)

    w_enc = jnp.concatenate([WencK.T, WencP1.T, WencP2.T], axis=1)
    b_enc = jnp.concatenate([bencK, bencP1, bencP2])[None, :]
    h1, h21, h22 = _encode(x, w_enc, b_enc)

    z = jnp.zeros((STRIPE, D), jnp.float32)
    agg, s = _sc1(src, dst, h1, h21, h22, z)

    u = _umlp(s, Wu0.T, bu0[None, :], Wu1.T, bu1[None, :], Wu2.T, bu2[None, :])

    usum = _sc2(src, u, z)

    agg3 = agg.reshape(NC, N_PAD, D)
    usum3 = usum.reshape(NC, N_PAD, D)
    return _kdot(agg3, usum3,
                 Wk0.T, bk0[None, :], Wk1.T, bk1[None, :], Wk2.T, bk2[None, :])


# speedup vs baseline: 6.3121x; 1.2148x over previous
import functools

import jax
import jax.numpy as jnp
from jax import lax
from jax.experimental import pallas as pl
from jax.experimental.pallas import tpu as pltpu
from jax.experimental.pallas import tpu_sc as plsc

N_NODES = 10000
N_EDGES = 320000
D = 128

NC = 2
NS = 16
NW = NC * NS
CH = 64
NCHT = N_EDGES // CH
NPAIRT = NCHT // 2
NPITER = 80
CHA = 128
NCHTA = N_EDGES // CHA
NPAIRA = NCHTA // 2
NPITERA = 40
N_PAD = 10240
STRIPE = N_PAD // NS

_MESH = plsc.VectorSubcoreMesh(core_axis_name="c", subcore_axis_name="s")


def _enc_body(x_ref, w_ref, b_ref, h1_ref, h21_ref, h22_ref):
    h = jnp.dot(x_ref[...], w_ref[...], preferred_element_type=jnp.float32)
    h = h + b_ref[...]
    h1_ref[...] = h[:, :D]
    h21_ref[...] = h[:, D:2 * D]
    h22_ref[...] = h[:, 2 * D:]


def _encode(x, w_enc, b_enc):
    rows = 2000
    grid = (N_NODES // rows,)
    out = jax.ShapeDtypeStruct((N_NODES, D), jnp.float32)
    return pl.pallas_call(
        _enc_body,
        grid=grid,
        in_specs=[
            pl.BlockSpec((rows, D), lambda i: (i, 0)),
            pl.BlockSpec((D, 3 * D), lambda i: (0, 0)),
            pl.BlockSpec((1, 3 * D), lambda i: (0, 0)),
        ],
        out_specs=[
            pl.BlockSpec((rows, D), lambda i: (i, 0)),
            pl.BlockSpec((rows, D), lambda i: (i, 0)),
            pl.BlockSpec((rows, D), lambda i: (i, 0)),
        ],
        out_shape=[out, out, out],
    )(x, w_enc, b_enc)


@functools.partial(
    pl.kernel,
    out_type=jax.ShapeDtypeStruct((N_EDGES, D), jnp.float32),
    mesh=_MESH,
    scratch_types=[
        pltpu.VMEM((2, CHA), jnp.int32),
        pltpu.VMEM((2, CHA), jnp.int32),
        pltpu.VMEM((CHA, D), jnp.float32),
        pltpu.VMEM((CHA, D), jnp.float32),
    ] + [pltpu.SemaphoreType.DMA] * 6,
)
def _sc1a(src_hbm, dst_hbm, h21_hbm, h22_hbm,
          s_hbm, idx_s, idx_d, srows_a, srows_b, si1, si2, s2, s4, s7, s8):
    c = lax.axis_index("c")
    sidx = lax.axis_index("s")
    wid = sidx * NC + c

    def body(j, carry):
        pid = j * NW + wid

        @pl.when(pid < NPAIRA)
        def _pair():
            i1 = pltpu.async_copy(src_hbm.at[pl.ds(2 * pid, 2)], idx_s, si1)
            i2 = pltpu.async_copy(dst_hbm.at[pl.ds(2 * pid, 2)], idx_d, si2)
            i1.wait()
            g2a = pltpu.async_copy(h21_hbm.at[idx_s.at[0]], srows_a, s2)
            g2b = pltpu.async_copy(h21_hbm.at[idx_s.at[1]], srows_b, s4)
            i2.wait()
            g2a.wait()
            g3a = pltpu.async_copy(h22_hbm.at[idx_d.at[0]], srows_a, s2,
                                   add=True)
            g2b.wait()
            g3b = pltpu.async_copy(h22_hbm.at[idx_d.at[1]], srows_b, s4,
                                   add=True)
            g3a.wait()
            wa = pltpu.async_copy(srows_a, s_hbm.at[pl.ds(2 * pid * CHA, CHA)],
                                  s7)
            g3b.wait()
            wb = pltpu.async_copy(srows_b,
                                  s_hbm.at[pl.ds((2 * pid + 1) * CHA, CHA)],
                                  s8)
            wa.wait()
            wb.wait()

        return carry

    lax.fori_loop(0, NPITERA, body, 0)


@functools.partial(
    pl.kernel,
    out_type=jax.ShapeDtypeStruct((NC * N_PAD, D), jnp.float32),
    mesh=_MESH,
    scratch_types=[
        pltpu.VMEM((2, CHA), jnp.int32),
        pltpu.VMEM((2, CHA), jnp.int32),
        pltpu.VMEM((CHA, D), jnp.float32),
        pltpu.VMEM((CHA, D), jnp.float32),
        pltpu.VMEM_SHARED((N_PAD, D), jnp.float32),
    ] + [pltpu.SemaphoreType.DMA] * 6,
)
def _sc1b(src_hbm, dst_hbm, h1_hbm, z_hbm,
          agg_hbm, idx_s, idx_d, rows_a, rows_b, aggsh,
          si1, si2, s1, s3, s5, s6):
    c = lax.axis_index("c")
    sidx = lax.axis_index("s")
    wid = sidx * NC + c
    tid = sidx

    pltpu.sync_copy(z_hbm, aggsh.at[pl.ds(tid * STRIPE, STRIPE)])
    plsc.subcore_barrier()

    def body(j, carry):
        pid = j * NW + wid

        @pl.when(pid < NPAIRA)
        def _pair():
            i1 = pltpu.async_copy(src_hbm.at[pl.ds(2 * pid, 2)], idx_s, si1)
            i2 = pltpu.async_copy(dst_hbm.at[pl.ds(2 * pid, 2)], idx_d, si2)
            i1.wait()
            g1a = pltpu.async_copy(h1_hbm.at[idx_s.at[0]], rows_a, s1)
            g1b = pltpu.async_copy(h1_hbm.at[idx_s.at[1]], rows_b, s3)
            i2.wait()
            g1a.wait()
            sca = pltpu.async_copy(rows_a, aggsh.at[idx_d.at[0]], s5,
                                   add=True)
            g1b.wait()
            scb = pltpu.async_copy(rows_b, aggsh.at[idx_d.at[1]], s6,
                                   add=True)
            sca.wait()
            scb.wait()

        return carry

    lax.fori_loop(0, NPITERA, body, 0)

    plsc.subcore_barrier()
    pltpu.sync_copy(aggsh.at[pl.ds(tid * STRIPE, STRIPE)],
                    agg_hbm.at[pl.ds(c * N_PAD + tid * STRIPE, STRIPE)])


def _umlp_body(s_ref, w0, b0, w1, b1, w2, b2, u_ref):
    h = jnp.tanh(jnp.dot(s_ref[...], w0[...],
                         preferred_element_type=jnp.float32) + b0[...])
    h = jnp.maximum(jnp.dot(h, w1[...],
                            preferred_element_type=jnp.float32) + b1[...], 0.0)
    u_ref[...] = jnp.dot(h, w2[...],
                         preferred_element_type=jnp.float32) + b2[...]


def _umlp(s, w0, b0, w1, b1, w2, b2):
    rows = 2000
    grid = (N_EDGES // rows,)
    wspec = pl.BlockSpec((D, D), lambda i: (0, 0))
    bspec = pl.BlockSpec((1, D), lambda i: (0, 0))
    return pl.pallas_call(
        _umlp_body,
        grid=grid,
        in_specs=[pl.BlockSpec((rows, D), lambda i: (i, 0)),
                  wspec, bspec, wspec, bspec, wspec, bspec],
        out_specs=pl.BlockSpec((rows, D), lambda i: (i, 0)),
        out_shape=jax.ShapeDtypeStruct((N_EDGES, D), jnp.float32),
    )(s, w0, b0, w1, b1, w2, b2)


@functools.partial(
    pl.kernel,
    out_type=jax.ShapeDtypeStruct((NC * N_PAD, D), jnp.float32),
    mesh=_MESH,
    scratch_types=[
        pltpu.VMEM((2, CH), jnp.int32),
        pltpu.VMEM((2, CH), jnp.int32),
        pltpu.VMEM((2 * CH, D), jnp.float32),
        pltpu.VMEM((2 * CH, D), jnp.float32),
        pltpu.VMEM_SHARED((N_PAD, D), jnp.float32),
    ] + [pltpu.SemaphoreType.DMA] * 6,
)
def _sc2(src_hbm, u_hbm, z_hbm, usum_hbm, idx_a, idx_b, rows_a, rows_b,
         ussh, si1, si2, sl1, sl2, sca, scb):
    c = lax.axis_index("c")
    sidx = lax.axis_index("s")
    wid = sidx * NC + c
    tid = sidx

    pltpu.sync_copy(z_hbm, ussh.at[pl.ds(tid * STRIPE, STRIPE)])
    plsc.subcore_barrier()

    def body(m, carry):
        pid_a = (2 * m) * NW + wid
        pid_b = (2 * m + 1) * NW + wid

        @pl.when(pid_a < NPAIRT)
        def _a():
            i_a = pltpu.async_copy(src_hbm.at[pl.ds(2 * pid_a, 2)], idx_a,
                                   si1)
            l_a = pltpu.async_copy(u_hbm.at[pl.ds(2 * pid_a * CH, 2 * CH)],
                                   rows_a, sl1)

            @pl.when(pid_b < NPAIRT)
            def _b():
                i_b = pltpu.async_copy(src_hbm.at[pl.ds(2 * pid_b, 2)],
                                       idx_b, si2)
                l_b = pltpu.async_copy(u_hbm.at[pl.ds(2 * pid_b * CH, 2 * CH)],
                                       rows_b, sl2)
                i_a.wait()
                l_a.wait()
                sa1 = pltpu.async_copy(rows_a.at[pl.ds(0, CH)],
                                       ussh.at[idx_a.at[0]], sca, add=True)
                sa2 = pltpu.async_copy(rows_a.at[pl.ds(CH, CH)],
                                       ussh.at[idx_a.at[1]], sca, add=True)
                i_b.wait()
                l_b.wait()
                sb1 = pltpu.async_copy(rows_b.at[pl.ds(0, CH)],
                                       ussh.at[idx_b.at[0]], scb, add=True)
                sb2 = pltpu.async_copy(rows_b.at[pl.ds(CH, CH)],
                                       ussh.at[idx_b.at[1]], scb, add=True)
                sa1.wait()
                sa2.wait()
                sb1.wait()
                sb2.wait()

            @pl.when(jnp.logical_not(pid_b < NPAIRT))
            def _a_only():
                i_a.wait()
                l_a.wait()
                sa1 = pltpu.async_copy(rows_a.at[pl.ds(0, CH)],
                                       ussh.at[idx_a.at[0]], sca, add=True)
                sa2 = pltpu.async_copy(rows_a.at[pl.ds(CH, CH)],
                                       ussh.at[idx_a.at[1]], sca, add=True)
                sa1.wait()
                sa2.wait()

        return carry

    lax.fori_loop(0, NPITER // 2, body, 0)

    plsc.subcore_barrier()
    pltpu.sync_copy(ussh.at[pl.ds(tid * STRIPE, STRIPE)],
                    usum_hbm.at[pl.ds(c * N_PAD + tid * STRIPE, STRIPE)])


def _kdot_body(agg_ref, usum_ref, w0, b0, w1, b1, w2, b2, out_ref):
    a = agg_ref[0] + agg_ref[1]
    us = usum_ref[0] + usum_ref[1]
    h = jnp.tanh(jnp.dot(a, w0[...],
                         preferred_element_type=jnp.float32) + b0[...])
    h = jnp.maximum(jnp.dot(h, w1[...],
                            preferred_element_type=jnp.float32) + b1[...], 0.0)
    k = jnp.dot(h, w2[...], preferred_element_type=jnp.float32) + b2[...]
    part = jnp.sum(k * us).reshape(1, 1)

    @pl.when(pl.program_id(0) == 0)
    def _():
        out_ref[...] = jnp.zeros((1, 1), jnp.float32)

    out_ref[...] += part


def _kdot(agg, usum, w0, b0, w1, b1, w2, b2):
    rows = 2048
    grid = (N_PAD // rows,)
    wspec = pl.BlockSpec((D, D), lambda i: (0, 0))
    bspec = pl.BlockSpec((1, D), lambda i: (0, 0))
    out = pl.pallas_call(
        _kdot_body,
        grid=grid,
        in_specs=[pl.BlockSpec((NC, rows, D), lambda i: (0, i, 0)),
                  pl.BlockSpec((NC, rows, D), lambda i: (0, i, 0)),
                  wspec, bspec, wspec, bspec, wspec, bspec],
        out_specs=pl.BlockSpec((1, 1), lambda i: (0, 0)),
        out_shape=jax.ShapeDtypeStruct((1, 1), jnp.float32),
    )(agg, usum, w0, b0, w1, b1, w2, b2)
    return out[0, 0]


def kernel(x, edge_index, e,
           Wk0, bk0, Wk1, bk1, Wk2, bk2,
           Wu0, bu0, Wu1, bu1, Wu2, bu2,
           WencK, bencK, WencP1, bencP1, WencP2, bencP2):
    src = edge_index[0].reshape(NCHT, CH)
    dst = edge_index[1].reshape(NCHT, CH)
    src_a = edge_index[0].reshape(NCHTA, CHA)
    dst_a = edge_index[1].reshape(NCHTA, CHA)

    w_enc = jnp.concatenate([WencK.T, WencP1.T, WencP2.T], axis=1)
    b_enc = jnp.concatenate([bencK, bencP1, bencP2])[None, :]
    h1, h21, h22 = _encode(x, w_enc, b_enc)

    z = jnp.zeros((STRIPE, D), jnp.float32)
    s = _sc1a(src_a, dst_a, h21, h22)
    agg = _sc1b(src_a, dst_a, h1, z)

    u = _umlp(s, Wu0.T, bu0[None, :], Wu1.T, bu1[None, :], Wu2.T, bu2[None, :])

    usum = _sc2(src, u, z)

    agg3 = agg.reshape(NC, N_PAD, D)
    usum3 = usum.reshape(NC, N_PAD, D)
    return _kdot(agg3, usum3,
                 Wk0.T, bk0[None, :], Wk1.T, bk1[None, :], Wk2.T, bk2[None, :])
